# baseline (speedup 1.0000x reference)
import functools

import jax
import jax.numpy as jnp
from jax import lax
from jax.experimental import pallas as pl
from jax.experimental.pallas import tpu as pltpu
from jax.experimental.pallas import tpu_sc as plsc

D_MODEL = 64
SCALE = 8.0
VOCAB = 1000000
B0 = 4096
B1 = 200
NUM_WORKERS = 32
N_TCOL = VOCAB // 128
TCOL_ITERS = 123
TAIL_COLS = VOCAB - N_TCOL * 128
TAIL_PAIRS = TAIL_COLS // 2

_mesh = plsc.VectorSubcoreMesh(core_axis_name="c", subcore_axis_name="s")

_IOTA = lambda: lax.iota(jnp.int32, 16)


@functools.partial(
    pl.kernel,
    mesh=_mesh,
    out_type=jax.ShapeDtypeStruct((VOCAB // 2, 128), jnp.float32),
    compiler_params=pltpu.CompilerParams(needs_layout_passes=False),
    scratch_types=[
        pltpu.VMEM((64, 128), jnp.float32),
        pltpu.VMEM((64, 128), jnp.float32),
        pltpu.VMEM((64, 128), jnp.float32),
        pltpu.VMEM((64, 128), jnp.float32),
        pltpu.SemaphoreType.DMA,
        pltpu.SemaphoreType.DMA,
        pltpu.SemaphoreType.DMA,
        pltpu.SemaphoreType.DMA,
    ],
)
def _pack_pairs(lut_t, tail_t, packed, s0, s1, w0, w1, is0, is1, os0, os1):
    wid = lax.axis_index("s") * 2 + lax.axis_index("c")
    sbufs = (s0, s1)
    wbufs = (w0, w1)
    isems = (is0, is1)
    osems = (os0, os1)

    def tcol(i):
        return i * NUM_WORKERS + wid

    def fire_in(i, b):
        @pl.when(tcol(i) < N_TCOL)
        def _():
            pltpu.async_copy(
                lut_t.at[:, pl.ds(tcol(i) * 128, 128)], sbufs[b], isems[b]
            )

    def wait_in(i, b):
        @pl.when(tcol(i) < N_TCOL)
        def _():
            pltpu.make_async_copy(
                lut_t.at[:, pl.ds(tcol(i) * 128, 128)], sbufs[b], isems[b]
            ).wait()

    def transpose(i, b):
        @pl.when(tcol(i) < N_TCOL)
        def _():
            s, w = sbufs[b], wbufs[b]
            for p in range(64):
                for kq in range(8):
                    hi = kq // 4
                    rowv = _IOTA() + (16 * kq - 64 * hi)
                    colv = jnp.full((16,), 2 * p + hi, jnp.int32)
                    w[p, pl.ds(16 * kq, 16)] = plsc.load_gather(s, [rowv, colv])

    def fire_out(i, b):
        @pl.when(tcol(i) < N_TCOL)
        def _():
            pltpu.async_copy(
                wbufs[b], packed.at[pl.ds(tcol(i) * 64, 64)], osems[b]
            )

    def wait_out(i, b):
        @pl.when(jnp.logical_and(i >= 0, tcol(i) < N_TCOL))
        def _():
            base = jnp.maximum(tcol(i), 0) * 64
            pltpu.make_async_copy(
                wbufs[b], packed.at[pl.ds(base, 64)], osems[b]
            ).wait()

    fire_in(0, 0)

    def body(it, carry):
        for h in range(2):
            i = 2 * it + h
            b = h
            nb = 1 - h
            fire_in(i + 1, nb)
            wait_in(i, b)
            wait_out(i - 2, b)
            transpose(i, b)
            fire_out(i, b)
        return carry

    lax.fori_loop(0, TCOL_ITERS, body, 0)
    wait_out(2 * TCOL_ITERS - 2, 0)
    wait_out(2 * TCOL_ITERS - 1, 1)

    @pl.when(wid == 0)
    def _():
        pltpu.sync_copy(tail_t, s0)
        for p in range(TAIL_PAIRS):
            for kq in range(8):
                hi = kq // 4
                rowv = _IOTA() + (16 * kq - 64 * hi)
                colv = jnp.full((16,), 64 + 2 * p + hi, jnp.int32)
                w0[p, pl.ds(16 * kq, 16)] = plsc.load_gather(s0, [rowv, colv])
        pltpu.sync_copy(
            w0.at[pl.ds(0, TAIL_PAIRS)],
            packed.at[pl.ds(N_TCOL * 64, TAIL_PAIRS)],
        )


@functools.partial(
    pl.kernel,
    mesh=_mesh,
    out_type=jax.ShapeDtypeStruct((B1, D_MODEL, B0), jnp.float32),
    compiler_params=pltpu.CompilerParams(needs_layout_passes=False),
    scratch_types=[
        pltpu.VMEM((B1, 128), jnp.int32),
        pltpu.VMEM((128,), jnp.int32),
        pltpu.VMEM((128,), jnp.int32),
        pltpu.VMEM((128, 128), jnp.float32),
        pltpu.VMEM((128, 128), jnp.float32),
        pltpu.VMEM((64, 128), jnp.float32),
        pltpu.VMEM((64, 128), jnp.float32),
        pltpu.SemaphoreType.DMA,
        pltpu.SemaphoreType.DMA,
        pltpu.SemaphoreType.DMA,
        pltpu.SemaphoreType.DMA,
    ],
)
def _gather_scale(x_t, packed, out_t, xcol, g0, g1, r0, r1, t0, t1,
                  gs0, gs1, os0, os1):
    wid = lax.axis_index("s") * 2 + lax.axis_index("c")
    gidx = (g0, g1)
    rbufs = (r0, r1)
    tbufs = (t0, t1)
    gsems = (gs0, gs1)
    osems = (os0, os1)

    pltpu.sync_copy(x_t.at[:, pl.ds(wid * 128, 128)], xcol)

    def prep_fire(b1, b):
        @pl.when(b1 < B1)
        def _():
            for k in range(8):
                gidx[b][pl.ds(16 * k, 16)] = (
                    xcol[b1, pl.ds(16 * k, 16)] >> 1
                )
            pltpu.async_copy(packed.at[gidx[b]], rbufs[b], gsems[b])

    def wait_g(b):
        pltpu.make_async_copy(packed.at[gidx[b]], rbufs[b], gsems[b]).wait()

    def extract(b1, b):
        r, t = rbufs[b], tbufs[b]
        for k in range(8):
            rowv = _IOTA() + 16 * k
            par64 = (xcol[b1, pl.ds(16 * k, 16)] & 1) * 64
            for d in range(D_MODEL):
                v = plsc.load_gather(r, [rowv, par64 + d])
                t[d, pl.ds(16 * k, 16)] = v * SCALE

    def fire_out(b1, b):
        pltpu.async_copy(
            tbufs[b], out_t.at[b1, :, pl.ds(wid * 128, 128)], osems[b]
        )

    def wait_out(b1, b):
        def do(bb):
            pltpu.make_async_copy(
                tbufs[b], out_t.at[bb, :, pl.ds(wid * 128, 128)], osems[b]
            ).wait()

        if isinstance(b1, int):
            do(b1)
        else:
            @pl.when(b1 >= 0)
            def _():
                do(jnp.maximum(b1, 0))

    prep_fire(0, 0)

    def body(it, carry):
        for h in range(2):
            b1 = 2 * it + h
            b = h
            nb = 1 - h
            prep_fire(b1 + 1, nb)
            wait_g(b)
            wait_out(b1 - 2, b)
            extract(b1, b)
            fire_out(b1, b)
        return carry

    lax.fori_loop(0, B1 // 2, body, 0)
    wait_out(B1 - 2, 0)
    wait_out(B1 - 1, 1)


def kernel(x, lut):
    packed = _pack_pairs(lut.T, lut[VOCAB - 128:].T)
    out_t = _gather_scale(x.T.astype(jnp.int32), packed)
    return jnp.transpose(out_t, (2, 0, 1))


# speedup vs baseline: 1.7513x; 1.7513x over previous
import functools

import jax
import jax.numpy as jnp
from jax import lax
from jax.experimental import pallas as pl
from jax.experimental.pallas import tpu as pltpu
from jax.experimental.pallas import tpu_sc as plsc

D_MODEL = 64
SCALE = 8.0
B0 = 4096
B1 = 200
B_TOTAL = B0 * B1
NUM_WORKERS = 32
PER_W = B_TOTAL // NUM_WORKERS
CHUNK = 512
N_CHUNKS = PER_W // CHUNK
IDX_W = 128
GPC = CHUNK // IDX_W
IDX_ROWS_PER_W = PER_W // IDX_W
HALF = CHUNK // 2
TMP_ROWS = B_TOTAL // 2

_mesh = plsc.VectorSubcoreMesh(core_axis_name="c", subcore_axis_name="s")


@functools.partial(
    pl.kernel,
    mesh=_mesh,
    out_type=jax.ShapeDtypeStruct((TMP_ROWS, 2 * D_MODEL), jnp.float32),
    compiler_params=pltpu.CompilerParams(use_tc_tiling_on_sc=False),
    scratch_types=[
        pltpu.VMEM((GPC, IDX_W), jnp.int32),
        pltpu.VMEM((GPC, IDX_W), jnp.int32),
        pltpu.VMEM((CHUNK, D_MODEL), jnp.float32),
        pltpu.VMEM((CHUNK, D_MODEL), jnp.float32),
        pltpu.SemaphoreType.DMA,
        pltpu.SemaphoreType.DMA,
        pltpu.SemaphoreType.DMA,
        pltpu.SemaphoreType.DMA,
    ],
)
def _gather(idx_hbm, lut_hbm, tmp_hbm, ib0, ib1, rb0, rb1,
            gs0, gs1, os0, os1):
    wid = lax.axis_index("s") * 2 + lax.axis_index("c")
    ibufs = (ib0, ib1)
    rbufs = (rb0, rb1)
    gsems = (gs0, gs1)
    osems = (os0, os1)

    def load_idx(g, b):
        row = wid * IDX_ROWS_PER_W + g * GPC
        pltpu.sync_copy(idx_hbm.at[pl.ds(row, GPC)], ibufs[b])

    def fire_gathers(b):
        for j in range(GPC):
            pltpu.async_copy(
                lut_hbm.at[ibufs[b].at[j]],
                rbufs[b].at[pl.ds(j * IDX_W, IDX_W)],
                gsems[b],
            )

    def wait_gathers(b):
        for j in range(GPC):
            pltpu.make_async_copy(
                lut_hbm.at[ibufs[b].at[j]],
                rbufs[b].at[pl.ds(j * IDX_W, IDX_W)],
                gsems[b],
            ).wait()

    def out_pairs(g, b):
        base = wid * (PER_W // 2) + g * HALF
        return (
            (rbufs[b].at[pl.ds(0, HALF)],
             tmp_hbm.at[pl.ds(base, HALF), pl.ds(0, D_MODEL)]),
            (rbufs[b].at[pl.ds(HALF, HALF)],
             tmp_hbm.at[pl.ds(base, HALF), pl.ds(D_MODEL, D_MODEL)]),
        )

    def fire_out(g, b):
        for src, dst in out_pairs(g, b):
            pltpu.async_copy(src, dst, osems[b])

    def wait_out(g, b):
        for src, dst in out_pairs(g, b):
            pltpu.make_async_copy(src, dst, osems[b]).wait()

    load_idx(0, 0)
    fire_gathers(0)
    for g in range(N_CHUNKS):
        b = g & 1
        nb = 1 - b
        if g + 1 < N_CHUNKS:
            load_idx(g + 1, nb)
            if g >= 1:
                wait_out(g - 1, nb)
            fire_gathers(nb)
        wait_gathers(b)
        fire_out(g, b)
    wait_out(N_CHUNKS - 2, (N_CHUNKS - 2) & 1)
    wait_out(N_CHUNKS - 1, (N_CHUNKS - 1) & 1)


def _xpose_body(t_ref, o_ref):
    blk = t_ref[...]
    o_ref[...] = (
        jnp.concatenate([blk[:, :D_MODEL].T, blk[:, D_MODEL:].T], axis=1)
        * SCALE
    )[None]


_N_B0_BLK = B0 // CHUNK


@jax.jit
def _xpose(tmp):
    return pl.pallas_call(
        _xpose_body,
        grid=(B1, _N_B0_BLK),
        in_specs=[
            pl.BlockSpec((HALF, 2 * D_MODEL), lambda i, j: (i * _N_B0_BLK + j, 0)),
        ],
        out_specs=pl.BlockSpec((1, D_MODEL, CHUNK), lambda i, j: (i, 0, j)),
        out_shape=jax.ShapeDtypeStruct((B1, D_MODEL, B0), jnp.float32),
    )(tmp)


def kernel(x, lut):
    xf = x.T.reshape(B_TOTAL // IDX_W, IDX_W).astype(jnp.int32)
    tmp = _gather(xf, lut)
    out_t = _xpose(tmp)
    return jnp.transpose(out_t, (2, 0, 1))


# speedup vs baseline: 2.9461x; 1.6822x over previous
import functools

import jax
import jax.numpy as jnp
from jax import lax
from jax.experimental import pallas as pl
from jax.experimental.pallas import tpu as pltpu
from jax.experimental.pallas import tpu_sc as plsc

D_MODEL = 64
SCALE = 8.0
B0 = 4096
B1 = 200
B_TOTAL = B0 * B1
NUM_WORKERS = 32
PER_W = B_TOTAL // NUM_WORKERS
CHUNK = 512
N_CHUNKS = PER_W // CHUNK
IDX_W = 128
GPC = CHUNK // IDX_W
IDX_ROWS_PER_W = PER_W // IDX_W
HALF = CHUNK // 2
TMP_ROWS = B_TOTAL // 2

_mesh = plsc.VectorSubcoreMesh(core_axis_name="c", subcore_axis_name="s")


@functools.partial(
    pl.kernel,
    mesh=_mesh,
    out_type=jax.ShapeDtypeStruct((TMP_ROWS, 2 * D_MODEL), jnp.float32),
    compiler_params=pltpu.CompilerParams(use_tc_tiling_on_sc=False),
    scratch_types=[
        pltpu.VMEM((GPC, IDX_W), jnp.int32),
        pltpu.VMEM((GPC, IDX_W), jnp.int32),
        pltpu.VMEM((CHUNK, D_MODEL), jnp.float32),
        pltpu.VMEM((CHUNK, D_MODEL), jnp.float32),
        pltpu.SemaphoreType.DMA,
        pltpu.SemaphoreType.DMA,
        pltpu.SemaphoreType.DMA,
        pltpu.SemaphoreType.DMA,
    ],
)
def _gather(idx_hbm, lut_hbm, tmp_hbm, ib0, ib1, rb0, rb1,
            gs0, gs1, os0, os1):
    wid = lax.axis_index("s") * 2 + lax.axis_index("c")
    ibufs = (ib0, ib1)
    rbufs = (rb0, rb1)
    gsems = (gs0, gs1)
    osems = (os0, os1)

    def load_idx(g, b):
        row = wid * IDX_ROWS_PER_W + g * GPC
        pltpu.sync_copy(idx_hbm.at[pl.ds(row, GPC)], ibufs[b])

    def fire_gathers(b):
        for j in range(GPC):
            pltpu.async_copy(
                lut_hbm.at[ibufs[b].at[j]],
                rbufs[b].at[pl.ds(j * IDX_W, IDX_W)],
                gsems[b],
            )

    def wait_gathers(b):
        for j in range(GPC):
            pltpu.make_async_copy(
                lut_hbm.at[ibufs[b].at[j]],
                rbufs[b].at[pl.ds(j * IDX_W, IDX_W)],
                gsems[b],
            ).wait()

    def out_pairs(g, b):
        base = wid * (PER_W // 2) + g * HALF
        return (
            (rbufs[b].at[pl.ds(0, HALF)],
             tmp_hbm.at[pl.ds(base, HALF), pl.ds(0, D_MODEL)]),
            (rbufs[b].at[pl.ds(HALF, HALF)],
             tmp_hbm.at[pl.ds(base, HALF), pl.ds(D_MODEL, D_MODEL)]),
        )

    def fire_out(g, b):
        for src, dst in out_pairs(g, b):
            pltpu.async_copy(src, dst, osems[b])

    def wait_out(g, b):
        for src, dst in out_pairs(g, b):
            pltpu.make_async_copy(src, dst, osems[b]).wait()

    load_idx(0, 0)
    fire_gathers(0)
    for g in range(N_CHUNKS):
        b = g & 1
        nb = 1 - b
        if g + 1 < N_CHUNKS:
            load_idx(g + 1, nb)
            if g >= 1:
                wait_out(g - 1, nb)
            fire_gathers(nb)
        wait_gathers(b)
        fire_out(g, b)
    wait_out(N_CHUNKS - 2, (N_CHUNKS - 2) & 1)
    wait_out(N_CHUNKS - 1, (N_CHUNKS - 1) & 1)


_N_B0_BLK = B0 // CHUNK
_ROWS_PER_B1 = B0 // 2


def _xpose_body(t_ref, o_ref):
    blk = t_ref[...]
    pieces = []
    for m in range(_N_B0_BLK):
        sub = blk[m * HALF:(m + 1) * HALF]
        pieces.append(sub[:, :D_MODEL].T)
        pieces.append(sub[:, D_MODEL:].T)
    o_ref[...] = (jnp.concatenate(pieces, axis=1) * SCALE)[None]


@jax.jit
def _xpose(tmp):
    return pl.pallas_call(
        _xpose_body,
        grid=(B1,),
        in_specs=[
            pl.BlockSpec((_ROWS_PER_B1, 2 * D_MODEL), lambda i: (i, 0)),
        ],
        out_specs=pl.BlockSpec((1, D_MODEL, B0), lambda i: (i, 0, 0)),
        out_shape=jax.ShapeDtypeStruct((B1, D_MODEL, B0), jnp.float32),
    )(tmp)


def kernel(x, lut):
    xf = x.T.reshape(B_TOTAL // IDX_W, IDX_W).astype(jnp.int32)
    tmp = _gather(xf, lut)
    out_t = _xpose(tmp)
    return jnp.transpose(out_t, (2, 0, 1))
